# baseline (speedup 1.0000x reference)
import functools

import numpy as np
import jax
import jax.numpy as jnp
from jax import lax
from jax.experimental import pallas as pl
from jax.experimental.pallas import tpu as pltpu
from jax.experimental.pallas import tpu_sc as plsc

POS_VOCAB = 128
LANES = 16


def _positions_np(n, vocab_size):
    lin = np.linspace(0.0, 1.0, n + 1, dtype=np.float32)
    intervals = np.stack([lin[:-1], lin[1:]]).T
    intervals = (intervals * vocab_size).astype(np.int32)
    intervals[:, 1] -= 1
    return np.round(intervals.astype(np.float32).mean(axis=-1)).astype(np.int32)


@functools.lru_cache(maxsize=None)
def _build(n_h, n_w, d, h_base, h_stride):
    info = plsc.get_sparse_core_info()
    nc, ns = info.num_cores, info.num_subcores
    assert n_h == nc * ns, "one subcore per output row"
    assert d % LANES == 0
    chunks = d // LANES
    mesh = plsc.VectorSubcoreMesh(core_axis_name="c", subcore_axis_name="s")

    @functools.partial(
        pl.kernel,
        mesh=mesh,
        out_type=jax.ShapeDtypeStruct((n_h * n_w, d), jnp.float32),
        scratch_types=[
            pltpu.VMEM((n_w,), jnp.int32),
            pltpu.VMEM((1, d), jnp.float32),
            pltpu.VMEM((n_w, d), jnp.float32),
            pltpu.SemaphoreType.DMA,
        ],
    )
    def pe_kernel(htab, wtab, wpos, out, widx_v, hrow_v, wrows_v, sem):
        wid = lax.axis_index("s") * nc + lax.axis_index("c")
        pltpu.sync_copy(wpos, widx_v)
        pltpu.async_copy(wtab.at[widx_v], wrows_v, sem).wait()
        hoff = h_base + h_stride * wid
        pltpu.sync_copy(htab.at[pl.ds(hoff, 1)], hrow_v)

        def body(c, carry):
            base = c * LANES
            h = hrow_v[0, pl.ds(base, LANES)]
            for j in range(n_w):
                wrows_v[j, pl.ds(base, LANES)] += h
            return carry

        lax.fori_loop(0, chunks, body, 0)
        pltpu.sync_copy(wrows_v, out.at[pl.ds(wid * n_w, n_w)])

    return pe_kernel


def kernel(x, height_table, width_table):
    n_h, n_w = x.shape[1], x.shape[2]
    d = height_table.shape[1]
    hpos = _positions_np(n_h, POS_VOCAB)
    wpos = _positions_np(n_w, POS_VOCAB)
    h_base = int(hpos[0])
    h_stride = int(hpos[1] - hpos[0]) if n_h > 1 else 0
    assert np.array_equal(hpos, h_base + h_stride * np.arange(n_h))
    pe = _build(n_h, n_w, d, h_base, h_stride)
    out = pe(height_table, width_table, jnp.asarray(wpos, dtype=jnp.int32))
    return out.reshape(n_h, n_w, d)


# speedup vs baseline: 1.2319x; 1.2319x over previous
import functools

import numpy as np
import jax
import jax.numpy as jnp
from jax import lax
from jax.experimental import pallas as pl
from jax.experimental.pallas import tpu as pltpu
from jax.experimental.pallas import tpu_sc as plsc

POS_VOCAB = 128


def _positions_np(n, vocab_size):
    lin = np.linspace(0.0, 1.0, n + 1, dtype=np.float32)
    intervals = np.stack([lin[:-1], lin[1:]]).T
    intervals = (intervals * vocab_size).astype(np.int32)
    intervals[:, 1] -= 1
    return np.round(intervals.astype(np.float32).mean(axis=-1)).astype(np.int32)


@functools.lru_cache(maxsize=None)
def _build_sc_gather(n_h, n_w, d, h_base, h_stride, w_base, w_stride):
    info = plsc.get_sparse_core_info()
    nc, ns = info.num_cores, info.num_subcores
    assert n_h == nc * ns and n_w == nc * ns
    mesh = plsc.VectorSubcoreMesh(core_axis_name="c", subcore_axis_name="s")

    @functools.partial(
        pl.kernel,
        mesh=mesh,
        out_type=(
            jax.ShapeDtypeStruct((n_h, d), jnp.float32),
            jax.ShapeDtypeStruct((n_w, d), jnp.float32),
        ),
        scratch_types=[
            pltpu.VMEM((1, d), jnp.float32),
            pltpu.VMEM((1, d), jnp.float32),
        ],
    )
    def gather_kernel(htab, wtab, hsel, wsel, hrow_v, wrow_v):
        wid = lax.axis_index("s") * nc + lax.axis_index("c")
        hoff = h_base + h_stride * wid
        woff = w_base + w_stride * wid
        pltpu.sync_copy(htab.at[pl.ds(hoff, 1)], hrow_v)
        pltpu.sync_copy(wtab.at[pl.ds(woff, 1)], wrow_v)
        pltpu.sync_copy(hrow_v, hsel.at[pl.ds(wid, 1)])
        pltpu.sync_copy(wrow_v, wsel.at[pl.ds(wid, 1)])

    return gather_kernel


def _tc_add_body(hsel_ref, wsel_ref, out_ref):
    out_ref[...] = hsel_ref[...][:, None, :] + wsel_ref[...][None, :, :]


@functools.lru_cache(maxsize=None)
def _build_tc_add(n_h, n_w, d):
    return pl.pallas_call(
        _tc_add_body,
        out_shape=jax.ShapeDtypeStruct((n_h, n_w, d), jnp.float32),
    )


def kernel(x, height_table, width_table):
    n_h, n_w = x.shape[1], x.shape[2]
    d = height_table.shape[1]
    hpos = _positions_np(n_h, POS_VOCAB)
    wpos = _positions_np(n_w, POS_VOCAB)
    h_base, h_stride = int(hpos[0]), int(hpos[1] - hpos[0]) if n_h > 1 else 0
    w_base, w_stride = int(wpos[0]), int(wpos[1] - wpos[0]) if n_w > 1 else 0
    assert np.array_equal(hpos, h_base + h_stride * np.arange(n_h))
    assert np.array_equal(wpos, w_base + w_stride * np.arange(n_w))
    hsel, wsel = _build_sc_gather(n_h, n_w, d, h_base, h_stride, w_base, w_stride)(
        height_table, width_table
    )
    return _build_tc_add(n_h, n_w, d)(hsel, wsel)
